# baseline (speedup 1.0000x reference)
import functools

import jax
import jax.numpy as jnp
from jax import lax
from jax.experimental import pallas as pl
from jax.experimental.pallas import tpu as pltpu
from jax.experimental.pallas import tpu_sc as plsc

SPELL_LEN = 9
HIDDEN = 128

NC = 2
NS = 16
NW = NC * NS
CH = 64
NBUF = 12


@functools.lru_cache(maxsize=None)
def _make(batch, rows):
    per_tile = batch // NW
    qch = per_tile // CH
    nch = SPELL_LEN * qch
    mesh = plsc.VectorSubcoreMesh(core_axis_name="c", subcore_axis_name="s")

    @functools.partial(
        pl.kernel,
        mesh=mesh,
        out_type=jax.ShapeDtypeStruct((SPELL_LEN, batch, HIDDEN), jnp.float32),
        compiler_params=pltpu.CompilerParams(use_tc_tiling_on_sc=True),
        scratch_types=[
            pltpu.VMEM((per_tile,), jnp.int32),
            pltpu.VMEM((per_tile * SPELL_LEN,), jnp.int32),
        ]
        + [pltpu.VMEM((CH, HIDDEN), jnp.float32) for _ in range(NBUF)]
        + [pltpu.SemaphoreType.DMA for _ in range(2 * NBUF)],
    )
    def k(rs_hbm, table_hbm, out_hbm, rs_v, idx_v, *rest):
        bufs = rest[:NBUF]
        gsems = rest[NBUF : 2 * NBUF]
        wsems = rest[2 * NBUF :]
        wid = lax.axis_index("s") * NC + lax.axis_index("c")
        base = wid * per_tile
        pltpu.sync_copy(rs_hbm.at[pl.ds(base, per_tile)], rs_v)

        for v in range(per_tile // 16):
            r9 = rs_v[pl.ds(v * 16, 16)] * SPELL_LEN
            for s in range(SPELL_LEN):
                idx_v[pl.ds(s * per_tile + v * 16, 16)] = r9 + s

        def gather(c, b):
            return pltpu.async_copy(
                table_hbm.at[idx_v.at[pl.ds(c * CH, CH)]],
                bufs[b],
                gsems[b],
            )

        def write(c, b):
            s, q = divmod(c, qch)
            return pltpu.async_copy(
                bufs[b],
                out_hbm.at[s, pl.ds(base + q * CH, CH)],
                wsems[b],
            )

        gdesc, wdesc = {}, {}
        depth = min(NBUF // 2, nch)
        for c in range(depth):
            gdesc[c % NBUF] = gather(c, c % NBUF)
        for c in range(nch):
            b = c % NBUF
            gdesc[b].wait()
            wdesc[b] = write(c, b)
            f = c + depth
            if f < nch:
                bf = f % NBUF
                if bf in wdesc:
                    wdesc[bf].wait()
                gdesc[bf] = gather(f, bf)
        for c in range(max(nch - NBUF, 0), nch):
            wdesc[c % NBUF].wait()

    return k


def kernel(rs_tensor, embedding_relation):
    batch = rs_tensor.shape[0]
    rows = embedding_relation.shape[0]
    out9 = _make(batch, rows)(rs_tensor, embedding_relation)
    return jnp.transpose(out9, (1, 0, 2))


# speedup vs baseline: 1.0222x; 1.0222x over previous
import functools

import jax
import jax.numpy as jnp
from jax import lax
from jax.experimental import pallas as pl
from jax.experimental.pallas import tpu as pltpu
from jax.experimental.pallas import tpu_sc as plsc

SPELL_LEN = 9
HIDDEN = 128

NC = 2
NS = 16
NW = NC * NS
CH = 128
NBUF = 6


@functools.lru_cache(maxsize=None)
def _make(batch, rows):
    per_tile = batch // NW
    qch = per_tile // CH
    nch = SPELL_LEN * qch
    mesh = plsc.VectorSubcoreMesh(core_axis_name="c", subcore_axis_name="s")

    @functools.partial(
        pl.kernel,
        mesh=mesh,
        out_type=jax.ShapeDtypeStruct((SPELL_LEN, batch, HIDDEN), jnp.float32),
        compiler_params=pltpu.CompilerParams(use_tc_tiling_on_sc=True),
        scratch_types=[
            pltpu.VMEM((per_tile,), jnp.int32),
            pltpu.VMEM((per_tile * SPELL_LEN,), jnp.int32),
        ]
        + [pltpu.VMEM((CH, HIDDEN), jnp.float32) for _ in range(NBUF)]
        + [pltpu.SemaphoreType.DMA for _ in range(2 * NBUF)],
    )
    def k(rs_hbm, table_hbm, out_hbm, rs_v, idx_v, *rest):
        bufs = rest[:NBUF]
        gsems = rest[NBUF : 2 * NBUF]
        wsems = rest[2 * NBUF :]
        wid = lax.axis_index("s") * NC + lax.axis_index("c")
        base = wid * per_tile
        pltpu.sync_copy(rs_hbm.at[pl.ds(base, per_tile)], rs_v)

        for v in range(per_tile // 16):
            r9 = rs_v[pl.ds(v * 16, 16)] * SPELL_LEN
            for s in range(SPELL_LEN):
                idx_v[pl.ds(s * per_tile + v * 16, 16)] = r9 + s

        def gather(c, b):
            return pltpu.async_copy(
                table_hbm.at[idx_v.at[pl.ds(c * CH, CH)]],
                bufs[b],
                gsems[b],
            )

        def write(c, b):
            s, q = divmod(c, qch)
            return pltpu.async_copy(
                bufs[b],
                out_hbm.at[s, pl.ds(base + q * CH, CH)],
                wsems[b],
            )

        gdesc, wdesc = {}, {}
        depth = min(NBUF - 1, nch)
        for c in range(depth):
            gdesc[c % NBUF] = gather(c, c % NBUF)
        for c in range(nch):
            b = c % NBUF
            gdesc[b].wait()
            wdesc[b] = write(c, b)
            f = c + depth
            if f < nch:
                bf = f % NBUF
                if bf in wdesc:
                    wdesc[bf].wait()
                gdesc[bf] = gather(f, bf)
        for c in range(max(nch - NBUF, 0), nch):
            wdesc[c % NBUF].wait()

    return k


def kernel(rs_tensor, embedding_relation):
    batch = rs_tensor.shape[0]
    rows = embedding_relation.shape[0]
    out9 = _make(batch, rows)(rs_tensor, embedding_relation)
    return jnp.transpose(out9, (1, 0, 2))
